# Initial kernel scaffold; baseline (speedup 1.0000x reference)
import jax
import jax.numpy as jnp
from jax.experimental import pallas as pl


def kernel(backbone_inputs, inputs, targets, features_bank):
    raise NotImplementedError("write your pallas kernel here")


# speedup vs baseline: 1.2251x; 1.2251x over previous
import jax
import jax.numpy as jnp
from jax.experimental import pallas as pl
from jax.experimental.pallas import tpu as pltpu

_B = 1024
_F = 32
_C = 100000
_INV_T = 20.0
_CHUNK = 2000
_NCHUNK = _C // _CHUNK


def _ce_kernel(inputs_ref, targets_ref, bank_ref, lse_ref, picked_ref,
               xn_ref, m_ref, s_ref, p_ref):
    c = pl.program_id(0)

    @pl.when(c == 0)
    def _init():
        x = inputs_ref[...]
        n2 = jnp.sum(x * x, axis=1, keepdims=True)
        xn_ref[...] = x / jnp.maximum(jnp.sqrt(n2), 1e-12)
        m_ref[...] = jnp.full((_B, 1), -1e30, jnp.float32)
        s_ref[...] = jnp.zeros((_B, 1), jnp.float32)
        p_ref[...] = jnp.zeros((_B, 1), jnp.float32)

    xn = xn_ref[...]
    chunk = bank_ref[...]
    logits = jax.lax.dot_general(
        xn, chunk, (((1,), (1,)), ((), ())),
        preferred_element_type=jnp.float32) * _INV_T

    col_ids = c * _CHUNK + jax.lax.broadcasted_iota(jnp.int32, (_B, _CHUNK), 1)
    hit = col_ids == targets_ref[...]
    p_ref[...] += jnp.sum(jnp.where(hit, logits, 0.0), axis=1, keepdims=True)

    m_old = m_ref[...]
    m_new = jnp.maximum(m_old, jnp.max(logits, axis=1, keepdims=True))
    s_ref[...] = (s_ref[...] * jnp.exp(m_old - m_new)
                  + jnp.sum(jnp.exp(logits - m_new), axis=1, keepdims=True))
    m_ref[...] = m_new

    @pl.when(c == _NCHUNK - 1)
    def _fin():
        lse_ref[...] = m_ref[...] + jnp.log(s_ref[...])
        picked_ref[...] = p_ref[...]


def kernel(backbone_inputs, inputs, targets, features_bank):
    del backbone_inputs
    tgt = targets.astype(jnp.int32).reshape(_B, 1)
    lse, picked = pl.pallas_call(
        _ce_kernel,
        grid=(_NCHUNK,),
        in_specs=[
            pl.BlockSpec((_B, _F), lambda c: (0, 0)),
            pl.BlockSpec((_B, 1), lambda c: (0, 0)),
            pl.BlockSpec((_CHUNK, _F), lambda c: (c, 0)),
        ],
        out_specs=[
            pl.BlockSpec((_B, 1), lambda c: (0, 0)),
            pl.BlockSpec((_B, 1), lambda c: (0, 0)),
        ],
        out_shape=[
            jax.ShapeDtypeStruct((_B, 1), jnp.float32),
            jax.ShapeDtypeStruct((_B, 1), jnp.float32),
        ],
        scratch_shapes=[
            pltpu.VMEM((_B, _F), jnp.float32),
            pltpu.VMEM((_B, 1), jnp.float32),
            pltpu.VMEM((_B, 1), jnp.float32),
            pltpu.VMEM((_B, 1), jnp.float32),
        ],
    )(inputs, tgt, features_bank)
    return jnp.mean(lse - picked)
